# baseline (device time: 48154 ns/iter reference)
import jax
import jax.numpy as jnp
from jax import lax
from jax.experimental import pallas as pl
from jax.experimental.pallas import tpu as pltpu

BF = jnp.bfloat16
C = 320
CS = 288
FC = 1024


def kernel(x, assign, W1, W2):
    t, d = x.shape
    e_per, _, f = W1.shape

    my_z = lax.axis_index("z")
    lo = (assign - 2 * my_z) % 4
    onehot = (lo[:, None] == jnp.arange(4, dtype=jnp.int32)[None, :]).astype(
        jnp.int32)
    rank = jnp.sum(onehot * (jnp.cumsum(onehot, axis=0) - 1), axis=1)
    slot = jnp.where(rank < C, lo * C + rank, 4 * C).astype(jnp.int32)
    slot_row = slot.reshape(1, t)
    slot_col = slot.reshape(t, 1)

    def body(x_hbm, sr_ref, sc_ref, w1_hbm, w2_hbm, out_ref,
             p_ref, pt_ref, xb_ref, ob_ref, xr_ref, ps_ref, pr_ref,
             w1_ref, w2_ref, xv_ref, send_sems, recv_sems, wsems):
        cx = pltpu.make_async_copy(x_hbm, xv_ref, wsems.at[4])
        cx.start()
        wcopies = []
        for e in range(e_per):
            c1 = pltpu.make_async_copy(w1_hbm.at[e], w1_ref.at[e],
                                       wsems.at[2 * e])
            c2 = pltpu.make_async_copy(w2_hbm.at[e], w2_ref.at[e],
                                       wsems.at[2 * e + 1])
            c1.start()
            c2.start()
            wcopies.append((c1, c2))
        mx = lax.axis_index("x")
        my = lax.axis_index("y")
        mz = lax.axis_index("z")
        nbr = (mx, my, 1 - mz)

        barrier_sem = pltpu.get_barrier_semaphore()
        pl.semaphore_signal(barrier_sem, inc=1, device_id=nbr,
                            device_id_type=pl.DeviceIdType.MESH)
        pl.semaphore_wait(barrier_sem, 1)

        cx.wait()
        x_bf = xv_ref[...].astype(BF)
        p_ref[...] = (lax.broadcasted_iota(jnp.int32, (4 * C, t), 0)
                      == sr_ref[...]).astype(BF)
        xb_ref[pl.ds(2 * C, 2 * C), :] = jnp.dot(
            p_ref[pl.ds(2 * C, 2 * C), :], x_bf,
            preferred_element_type=jnp.float32).astype(BF)
        rdma_x = []
        for b in range(2):
            r = pltpu.make_async_remote_copy(
                src_ref=xb_ref.at[pl.ds((2 + b) * C, CS)],
                dst_ref=xr_ref.at[pl.ds(b * C, CS)],
                send_sem=send_sems.at[b], recv_sem=recv_sems.at[b],
                device_id=nbr, device_id_type=pl.DeviceIdType.MESH)
            r.start()
            rdma_x.append(r)
        pt_ref[...] = (lax.broadcasted_iota(jnp.int32, (t, 4 * C), 1)
                       == sc_ref[...]).astype(BF)
        xb_ref[pl.ds(0, 2 * C), :] = jnp.dot(
            p_ref[pl.ds(0, 2 * C), :], x_bf,
            preferred_element_type=jnp.float32).astype(BF)
        zer = jnp.zeros((C - CS, d), BF)
        for b in range(2):
            ob_ref[pl.ds(b * C + CS, C - CS), :] = zer
            pr_ref[pl.ds(b * C + CS, C - CS), :] = zer

        def ffn(xin_bf, e, n=C):
            xin = xin_bf.astype(jnp.float32)
            y = jnp.zeros((n, d), jnp.float32)
            for fc in range(0, f, FC):
                h = jnp.maximum(
                    jnp.dot(xin, w1_ref[e, :, pl.ds(fc, FC)],
                            preferred_element_type=jnp.float32),
                    0.0)
                y = y + jnp.dot(h, w2_ref[e, pl.ds(fc, FC), :],
                                preferred_element_type=jnp.float32)
            return y

        rdma_p = []

        def send_partial(row0, nrows, sem_i):
            r = pltpu.make_async_remote_copy(
                src_ref=ps_ref.at[pl.ds(row0, nrows)],
                dst_ref=pr_ref.at[pl.ds(row0, nrows)],
                send_sem=send_sems.at[sem_i], recv_sem=recv_sems.at[sem_i],
                device_id=nbr, device_id_type=pl.DeviceIdType.MESH)
            r.start()
            rdma_p.append(r)

        H = CS // 2
        for e in range(e_per):
            wcopies[e][0].wait()
            wcopies[e][1].wait()
            rows = pl.ds(e * C, CS)
            ob_ref[rows, :] = ffn(xb_ref[rows, :], e, n=CS).astype(BF)
            rdma_x[e].wait()
            if e == 0:
                ps_ref[rows, :] = ffn(xr_ref[rows, :], e, n=CS).astype(BF)
                send_partial(0, CS, 2)
            else:
                for hh in range(2):
                    hrows = pl.ds(C + hh * H, H)
                    ps_ref[hrows, :] = ffn(xr_ref[hrows, :], e, n=H).astype(BF)
                    send_partial(C + hh * H, H, 3 + hh)

        partA = jnp.dot(pt_ref[:, pl.ds(0, 2 * C)], ob_ref[...],
                        preferred_element_type=jnp.float32)
        for r in rdma_p:
            r.wait()
        out_ref[...] = (partA + jnp.dot(
            pt_ref[:, pl.ds(2 * C, 2 * C)], pr_ref[...],
            preferred_element_type=jnp.float32)).astype(BF)

    return pl.pallas_call(
        body,
        out_shape=jax.ShapeDtypeStruct((t, d), BF),
        in_specs=[pl.BlockSpec(memory_space=pltpu.MemorySpace.HBM)]
        + [pl.BlockSpec(memory_space=pltpu.VMEM)] * 2
        + [pl.BlockSpec(memory_space=pltpu.MemorySpace.HBM)] * 2,
        out_specs=pl.BlockSpec(memory_space=pltpu.VMEM),
        scratch_shapes=[
            pltpu.VMEM((4 * C, t), BF),
            pltpu.VMEM((t, 4 * C), BF),
            pltpu.VMEM((4 * C, d), BF),
            pltpu.VMEM((2 * C, d), BF),
            pltpu.VMEM((2 * C, d), BF),
            pltpu.VMEM((2 * C, d), BF),
            pltpu.VMEM((2 * C, d), BF),
            pltpu.VMEM((e_per, d, f), jnp.float32),
            pltpu.VMEM((e_per, f, d), jnp.float32),
            pltpu.VMEM((t, d), jnp.float32),
            pltpu.SemaphoreType.DMA((5,)),
            pltpu.SemaphoreType.DMA((5,)),
            pltpu.SemaphoreType.DMA((5,)),
        ],
        compiler_params=pltpu.CompilerParams(
            collective_id=0, vmem_limit_bytes=100 * 1024 * 1024),
    )(x, slot_row, slot_col, W1, W2)


# device time: 47035 ns/iter; 1.0238x vs baseline; 1.0238x over previous
import jax
import jax.numpy as jnp
from jax import lax
from jax.experimental import pallas as pl
from jax.experimental.pallas import tpu as pltpu

BF = jnp.bfloat16
C = 320
CS = 288
FC = 1024


def kernel(x, assign, W1, W2):
    t, d = x.shape
    e_per, _, f = W1.shape

    my_z = lax.axis_index("z")
    lo = (assign - 2 * my_z) % 4
    onehot = (lo[:, None] == jnp.arange(4, dtype=jnp.int32)[None, :]).astype(
        jnp.int32)
    rank = jnp.sum(onehot * (jnp.cumsum(onehot, axis=0) - 1), axis=1)
    slot = jnp.where(rank < C, lo * C + rank, 4 * C).astype(jnp.int32)
    slot_row = slot.reshape(1, t)
    slot_col = slot.reshape(t, 1)

    def body(x_ref, sr_ref, sc_ref, w1_hbm, w2_hbm, out_ref,
             p_ref, pt_ref, xb_ref, ob_ref, xr_ref, ps_ref, pr_ref,
             w1_ref, w2_ref, send_sems, recv_sems, wsems):
        wcopies = []
        for e in range(e_per):
            c1 = pltpu.make_async_copy(w1_hbm.at[e], w1_ref.at[e],
                                       wsems.at[2 * e])
            c2 = pltpu.make_async_copy(w2_hbm.at[e], w2_ref.at[e],
                                       wsems.at[2 * e + 1])
            c1.start()
            c2.start()
            wcopies.append((c1, c2))
        mx = lax.axis_index("x")
        my = lax.axis_index("y")
        mz = lax.axis_index("z")
        nbr = (mx, my, 1 - mz)

        barrier_sem = pltpu.get_barrier_semaphore()
        pl.semaphore_signal(barrier_sem, inc=1, device_id=nbr,
                            device_id_type=pl.DeviceIdType.MESH)
        pl.semaphore_wait(barrier_sem, 1)

        x_bf = x_ref[...].astype(BF)
        p_ref[...] = (lax.broadcasted_iota(jnp.int32, (4 * C, t), 0)
                      == sr_ref[...]).astype(BF)
        xb_ref[pl.ds(2 * C, 2 * C), :] = jnp.dot(
            p_ref[pl.ds(2 * C, 2 * C), :], x_bf,
            preferred_element_type=jnp.float32).astype(BF)
        rdma_x = []
        for b in range(2):
            r = pltpu.make_async_remote_copy(
                src_ref=xb_ref.at[pl.ds((2 + b) * C, CS)],
                dst_ref=xr_ref.at[pl.ds(b * C, CS)],
                send_sem=send_sems.at[b], recv_sem=recv_sems.at[b],
                device_id=nbr, device_id_type=pl.DeviceIdType.MESH)
            r.start()
            rdma_x.append(r)
        pt_ref[...] = (lax.broadcasted_iota(jnp.int32, (t, 4 * C), 1)
                       == sc_ref[...]).astype(BF)
        xb_ref[pl.ds(0, 2 * C), :] = jnp.dot(
            p_ref[pl.ds(0, 2 * C), :], x_bf,
            preferred_element_type=jnp.float32).astype(BF)
        zer = jnp.zeros((C - CS, d), BF)
        for b in range(2):
            ob_ref[pl.ds(b * C + CS, C - CS), :] = zer
            pr_ref[pl.ds(b * C + CS, C - CS), :] = zer

        def ffn(xin_bf, e, n=C):
            xin = xin_bf.astype(jnp.float32)
            y = jnp.zeros((n, d), jnp.float32)
            for fc in range(0, f, FC):
                h = jnp.maximum(
                    jnp.dot(xin, w1_ref[e, :, pl.ds(fc, FC)],
                            preferred_element_type=jnp.float32),
                    0.0)
                y = y + jnp.dot(h, w2_ref[e, pl.ds(fc, FC), :],
                                preferred_element_type=jnp.float32)
            return y

        rdma_p = []

        def send_partial(row0, nrows, sem_i):
            r = pltpu.make_async_remote_copy(
                src_ref=ps_ref.at[pl.ds(row0, nrows)],
                dst_ref=pr_ref.at[pl.ds(row0, nrows)],
                send_sem=send_sems.at[sem_i], recv_sem=recv_sems.at[sem_i],
                device_id=nbr, device_id_type=pl.DeviceIdType.MESH)
            r.start()
            rdma_p.append(r)

        H = CS // 2
        for e in range(e_per):
            wcopies[e][0].wait()
            wcopies[e][1].wait()
            rows = pl.ds(e * C, CS)
            ob_ref[rows, :] = ffn(xb_ref[rows, :], e, n=CS).astype(BF)
            rdma_x[e].wait()
            if e == 0:
                ps_ref[rows, :] = ffn(xr_ref[rows, :], e, n=CS).astype(BF)
                send_partial(0, CS, 2)
            else:
                for hh in range(2):
                    hrows = pl.ds(C + hh * H, H)
                    ps_ref[hrows, :] = ffn(xr_ref[hrows, :], e, n=H).astype(BF)
                    send_partial(C + hh * H, H, 3 + hh)

        partA = jnp.dot(pt_ref[:, pl.ds(0, 2 * C)], ob_ref[...],
                        preferred_element_type=jnp.float32)
        for r in rdma_p:
            r.wait()
        out_ref[...] = (partA + jnp.dot(
            pt_ref[:, pl.ds(2 * C, 2 * C)], pr_ref[...],
            preferred_element_type=jnp.float32)).astype(BF)

    return pl.pallas_call(
        body,
        out_shape=jax.ShapeDtypeStruct((t, d), BF),
        in_specs=[pl.BlockSpec(memory_space=pltpu.VMEM)] * 3
        + [pl.BlockSpec(memory_space=pltpu.MemorySpace.HBM)] * 2,
        out_specs=pl.BlockSpec(memory_space=pltpu.VMEM),
        scratch_shapes=[
            pltpu.VMEM((4 * C, t), BF),
            pltpu.VMEM((t, 4 * C), BF),
            pltpu.VMEM((4 * C, d), BF),
            pltpu.VMEM((2 * C, d), BF),
            pltpu.VMEM((2 * C, d), BF),
            pltpu.VMEM((2 * C, d), BF),
            pltpu.VMEM((2 * C, d), BF),
            pltpu.VMEM((e_per, d, f), jnp.float32),
            pltpu.VMEM((e_per, f, d), jnp.float32),
            pltpu.SemaphoreType.DMA((5,)),
            pltpu.SemaphoreType.DMA((5,)),
            pltpu.SemaphoreType.DMA((4,)),
        ],
        compiler_params=pltpu.CompilerParams(
            collective_id=0, vmem_limit_bytes=100 * 1024 * 1024),
    )(x, slot_row, slot_col, W1, W2)
